# baseline (speedup 1.0000x reference)
import functools
import math

import jax
import jax.numpy as jnp
from jax import lax
from jax.experimental import pallas as pl
from jax.experimental.pallas import tpu as pltpu
from jax.experimental.pallas import tpu_sc as plsc

D_MODEL = 128
SCALE = math.sqrt(float(D_MODEL))

_NC = 2
_NS = 16
_NW = _NC * _NS

N_SPLIT = 4
GRP = 4
NBUF = 4
LEAD = 2


def _make_sc_gather(rows, S, D):
    assert rows % (_NW * GRP) == 0
    rpw = rows // _NW
    gpw = rpw // GRP

    mesh = plsc.VectorSubcoreMesh(core_axis_name="c", subcore_axis_name="s")

    @functools.partial(
        pl.kernel,
        mesh=mesh,
        out_type=jax.ShapeDtypeStruct((rows // GRP, GRP * S, D), jnp.float32),
        scratch_types=[
            pltpu.VMEM((rpw, S), jnp.int32),
            *([pltpu.VMEM((GRP * S, D), jnp.float32)] * NBUF),
            *([pltpu.SemaphoreType.DMA] * NBUF),
            *([pltpu.SemaphoreType.DMA] * NBUF),
        ],
    )
    def sc_gather(table_hbm, idx_hbm, y_hbm, idx_v, *bufs):
        buf = bufs[:NBUF]
        gsem = bufs[NBUF:2 * NBUF]
        ssem = bufs[2 * NBUF:3 * NBUF]

        wid = lax.axis_index("s") * _NC + lax.axis_index("c")
        pltpu.sync_copy(idx_hbm.at[wid], idx_v)
        grp0 = wid * gpw

        def issue_gathers(g):
            b = g % NBUF
            for j in range(GRP):
                pltpu.make_async_copy(
                    table_hbm.at[idx_v.at[g * GRP + j]],
                    buf[b].at[pl.ds(j * S, S)], gsem[b]).start()

        def wait_gathers(g):
            b = g % NBUF
            for j in range(GRP):
                pltpu.make_async_copy(
                    table_hbm.at[idx_v.at[0]],
                    buf[b].at[pl.ds(j * S, S)], gsem[b]).wait()

        def issue_scatter(g):
            b = g % NBUF
            pltpu.make_async_copy(buf[b], y_hbm.at[grp0 + g], ssem[b]).start()

        def wait_scatter(g):
            b = g % NBUF
            pltpu.make_async_copy(buf[b], y_hbm.at[0], ssem[b]).wait()

        for g in range(LEAD):
            issue_gathers(g)
        for g in range(gpw):
            wait_gathers(g)
            issue_scatter(g)
            nxt = g + LEAD
            if nxt < gpw:
                if nxt >= NBUF:
                    wait_scatter(nxt - NBUF)
                issue_gathers(nxt)
        for g in range(gpw - NBUF, gpw):
            wait_scatter(g)

    return sc_gather


def _tc_relayout(y, out_prev, k, rows, S, D):
    blk = 8 // GRP

    def body(y_ref, prev_ref, o_ref):
        for g in range(blk):
            for j in range(GRP):
                o_ref[g * GRP + j] = y_ref[g, pl.ds(j * S, S), :] * SCALE

    grid = (rows // 8,)
    y_spec = pl.BlockSpec((blk, GRP * S, D), lambda i: (i, 0, 0))
    prev_spec = pl.BlockSpec(memory_space=pl.ANY)
    o_spec = pl.BlockSpec((8, S, D), lambda i: (k * rows // 8 + i, 0, 0))
    return pl.pallas_call(
        body,
        grid=grid,
        in_specs=[y_spec, prev_spec],
        out_specs=o_spec,
        out_shape=jax.ShapeDtypeStruct((N_SPLIT * rows, S, D), jnp.float32),
        input_output_aliases={1: 0},
    )(y, out_prev)


def _tc_relayout_first(y, rows, S, D):
    blk = 8 // GRP

    def body(y_ref, o_ref):
        for g in range(blk):
            for j in range(GRP):
                o_ref[g * GRP + j] = y_ref[g, pl.ds(j * S, S), :] * SCALE

    grid = (rows // 8,)
    y_spec = pl.BlockSpec((blk, GRP * S, D), lambda i: (i, 0, 0))
    o_spec = pl.BlockSpec((8, S, D), lambda i: (i, 0, 0))
    return pl.pallas_call(
        body,
        grid=grid,
        in_specs=[y_spec],
        out_specs=o_spec,
        out_shape=jax.ShapeDtypeStruct((N_SPLIT * rows, S, D), jnp.float32),
    )(y)


def kernel(x, target_vec, table, W, b):
    B, S = x.shape
    V, D = table.shape
    bc = B // N_SPLIT
    idx = x.reshape(N_SPLIT, _NW, bc // _NW, S).astype(jnp.int32)
    sc_gather = _make_sc_gather(bc, S, D)
    out = None
    for k in range(N_SPLIT):
        y = sc_gather(table, idx[k])
        if k == 0:
            out = _tc_relayout_first(y, bc, S, D)
        else:
            out = _tc_relayout(y, out, k, bc, S, D)
    return out
